# baseline (speedup 1.0000x reference)
import functools

import jax
import jax.numpy as jnp
from jax import lax
from jax.experimental import pallas as pl
from jax.experimental.pallas import tpu as pltpu

try:
    from jax.experimental.pallas import tpu_sc as plsc
    _HAS_SC = True
except ImportError:
    plsc = None
    _HAS_SC = False

_SC_TILES = 32
_SC_RECS = 8


def _p1_body(x_ref, ssq_ref, win_ref, *, kz):
    k = pl.program_id(1)
    t = x_ref[0]
    s = jnp.sum(t * t, axis=0)
    s = jnp.sum(s, axis=0)
    iz = lax.broadcasted_iota(jnp.int32, s.shape, 0)
    iy = lax.broadcasted_iota(jnp.int32, s.shape, 1) % 2
    ix = lax.broadcasted_iota(jnp.int32, s.shape, 2) % 2
    lane = lax.broadcasted_iota(jnp.int32, (1, 8), 1)
    vec = jnp.zeros((1, 8), jnp.float32)
    for p in range(8):
        pz, px, py = (p >> 2) & 1, (p >> 1) & 1, p & 1
        m = (iz == pz) & (iy == py) & (ix == px)
        v = jnp.sum(jnp.where(m, s, 0.0))
        vec = vec + jnp.where(lane == p, v, 0.0)

    @pl.when(k == 0)
    def _():
        ssq_ref[0] = vec

    @pl.when(k != 0)
    def _():
        ssq_ref[0] = ssq_ref[0] + vec

    @pl.when(k == kz - 1)
    def _():
        a = ssq_ref[0]
        mx = jnp.max(a)
        idx = jnp.min(jnp.where(a >= mx, lane, 8))
        win_ref[0] = jnp.zeros((1, 8), jnp.int32) + idx


def _p2_body(w_ref, x_ref, o_ref):
    b = pl.program_id(0)
    p = w_ref[b]
    pz = p // 4
    px = (p // 2) % 2
    py = p % 2
    t = x_ref[0, :, :, pz]
    c_, zc2, yy, xx = t.shape
    ixs = 2 * lax.broadcasted_iota(jnp.int32, (c_, zc2, yy, xx // 2), 3) + px
    a = jnp.take_along_axis(t, ixs, axis=3)
    a = jnp.swapaxes(a, -1, -2)
    iy = 2 * lax.broadcasted_iota(jnp.int32, (c_, zc2, xx // 2, yy // 2), 3) + py
    a = jnp.take_along_axis(a, iy, axis=3)
    o_ref[0] = jnp.swapaxes(a, -1, -2)


def _sc_pass2(xplanes, w2d, B, C, Zh, Yh, X):
    n_rec = B * C * Zh
    per_tile = n_rec // _SC_TILES
    nch = per_tile // _SC_RECS
    Xh = X // 2
    rec_per_b = C * Zh
    plane = 2 * Yh * X
    out_rows_chunk = _SC_RECS * Yh
    mesh = plsc.VectorSubcoreMesh(core_axis_name="c", subcore_axis_name="s")

    @functools.partial(
        pl.kernel,
        out_type=jax.ShapeDtypeStruct((B * C * Zh * Yh, Xh), jnp.float32),
        mesh=mesh,
        scratch_types=[
            pltpu.VMEM((16,), jnp.int32),
            pltpu.VMEM((per_tile,), jnp.int32),
            pltpu.VMEM((_SC_RECS, plane), jnp.float32),
            pltpu.VMEM((_SC_RECS, plane), jnp.float32),
            pltpu.VMEM((out_rows_chunk, Xh), jnp.float32),
            pltpu.SemaphoreType.DMA,
            pltpu.SemaphoreType.DMA,
        ],
        compiler_params=pltpu.CompilerParams(needs_layout_passes=False),
    )
    def sc_fn(x_hbm, w_hbm, out_hbm, w_v, idx_v, rows_v0, rows_v1, out_v,
              sem0, sem1):
        wid = lax.axis_index("s") * 2 + lax.axis_index("c")
        pltpu.sync_copy(w_hbm.at[wid], w_v)
        iota = lax.iota(jnp.int32, 16)
        zero16 = jnp.zeros((16,), jnp.int32)
        b_t = (wid * per_tile) // rec_per_b
        wvec = w_v[...]
        pz = (wvec >> 2) & 1
        px = (wvec >> 1) & 1
        py = wvec & 1

        lzh = Zh.bit_length() - 1

        def gidx(g, c2):
            q = zero16 + wid * per_tile + g * 16 + iota
            zi = q & (Zh - 1)
            cc = (q >> lzh) & (C - 1)
            r = ((zero16 + b_t) * C + cc) * (2 * Zh) + 2 * zi + pz
            idx_v[pl.ds(g * 16, 16)] = r
            return c2

        lax.fori_loop(0, per_tile // 16, gidx, 0)

        def fire(ch, buf, sem):
            return pltpu.async_copy(
                x_hbm.at[idx_v.at[pl.ds(ch * _SC_RECS, _SC_RECS)]], buf, sem)

        def drain(buf, sem):
            pltpu.make_async_copy(
                x_hbm.at[idx_v.at[pl.ds(0, _SC_RECS)]], buf, sem).wait()

        def emit(ch, buf):
            def comp(i, c3):
                rec = zero16 + (i >> 5)
                yi = i & (Yh - 1)
                for h in range(Xh // 16):
                    cols = yi * (2 * X) + py * X + 2 * (iota + h * 16) + px
                    v = plsc.load_gather(buf, [rec, cols])
                    out_v[i, pl.ds(h * 16, 16)] = v
                return c3

            lax.fori_loop(0, out_rows_chunk, comp, 0)
            base = (wid * per_tile + ch * _SC_RECS) * Yh
            pltpu.sync_copy(out_v, out_hbm.at[pl.ds(base, out_rows_chunk)])

        fire(0, rows_v0, sem0)

        def pair(ch2, carry):
            ch = 2 * ch2
            fire(ch + 1, rows_v1, sem1)
            drain(rows_v0, sem0)
            emit(ch, rows_v0)
            fire(ch + 2, rows_v0, sem0)
            drain(rows_v1, sem1)
            emit(ch + 1, rows_v1)
            return carry

        lax.fori_loop(0, nch // 2 - 1, pair, 0)
        fire(nch - 1, rows_v1, sem1)
        drain(rows_v0, sem0)
        emit(nch - 2, rows_v0)
        drain(rows_v1, sem1)
        emit(nch - 1, rows_v1)

    return sc_fn(xplanes, w2d)


def kernel(input_to_pool):
    xin = input_to_pool
    B, C, Z, Y, X = xin.shape
    Zh, Yh = Z // 2, Y // 2
    x6 = xin.reshape(B, C, Zh, 2, Y, X)

    Zc = 4 if Zh % 4 == 0 else 1
    KZ = Zh // Zc

    ssq, win = pl.pallas_call(
        lambda xr, sr, wr: _p1_body(xr, sr, wr, kz=KZ),
        grid=(B, KZ),
        in_specs=[pl.BlockSpec((1, C, Zc, 2, Y, X),
                               lambda b, k: (b, 0, k, 0, 0, 0))],
        out_specs=[
            pl.BlockSpec((1, 1, 8), lambda b, k: (b, 0, 0)),
            pl.BlockSpec((1, 1, 8), lambda b, k: (b, 0, 0)),
        ],
        out_shape=[
            jax.ShapeDtypeStruct((B, 1, 8), jnp.float32),
            jax.ShapeDtypeStruct((B, 1, 8), jnp.int32),
        ],
        compiler_params=pltpu.CompilerParams(
            dimension_semantics=("parallel", "arbitrary"),
        ),
    )(x6)

    w = win[:, 0, 0]

    n_rec = B * C * Zh
    per_tile = n_rec // _SC_TILES
    pow2 = lambda v: v & (v - 1) == 0
    use_sc = (
        _HAS_SC
        and n_rec % (_SC_TILES * _SC_RECS * 2) == 0
        and (C * Zh) % per_tile == 0
        and per_tile % 16 == 0
        and X % 32 == 0
        and pow2(Zh) and pow2(C) and pow2(Yh) and pow2(X)
        and B <= 16
    )
    if use_sc:
        xplanes = xin.reshape(B * C * Z, Y * X)
        tile_b = jnp.arange(_SC_TILES) * per_tile // (C * Zh)
        wtab = jnp.broadcast_to(w[tile_b][:, None], (_SC_TILES, 16))
        out_flat = _sc_pass2(xplanes, wtab, B, C, Zh, Yh, X)
        return out_flat.reshape(B, C, Zh, Yh, X // 2)

    Zc2 = 4 if Zh % 4 == 0 else 1
    KZ2 = Zh // Zc2

    grid_spec = pltpu.PrefetchScalarGridSpec(
        num_scalar_prefetch=1,
        grid=(B, KZ2),
        in_specs=[
            pl.BlockSpec((1, C, Zc2, 2, Y, X),
                         lambda b, k, wr: (b, 0, k, 0, 0, 0)),
        ],
        out_specs=pl.BlockSpec((1, C, Zc2, Yh, X // 2),
                               lambda b, k, wr: (b, 0, k, 0, 0)),
    )
    out = pl.pallas_call(
        _p2_body,
        grid_spec=grid_spec,
        out_shape=jax.ShapeDtypeStruct((B, C, Zh, Yh, X // 2), jnp.float32),
        compiler_params=pltpu.CompilerParams(
            dimension_semantics=("parallel", "parallel"),
        ),
    )(w, x6)
    return out


# speedup vs baseline: 1.4811x; 1.4811x over previous
import jax
import jax.numpy as jnp
from jax import lax
from jax.experimental import pallas as pl
from jax.experimental.pallas import tpu as pltpu

def _p1_body(x_ref, ssq_ref, win_ref, *, kz):
    k = pl.program_id(1)
    t = x_ref[0]
    s = jnp.sum(t * t, axis=0)
    s = jnp.sum(s, axis=0)
    iz = lax.broadcasted_iota(jnp.int32, s.shape, 0)
    iy = lax.broadcasted_iota(jnp.int32, s.shape, 1) % 2
    ix = lax.broadcasted_iota(jnp.int32, s.shape, 2) % 2
    lane = lax.broadcasted_iota(jnp.int32, (1, 8), 1)
    vec = jnp.zeros((1, 8), jnp.float32)
    for p in range(8):
        pz, px, py = (p >> 2) & 1, (p >> 1) & 1, p & 1
        m = (iz == pz) & (iy == py) & (ix == px)
        v = jnp.sum(jnp.where(m, s, 0.0))
        vec = vec + jnp.where(lane == p, v, 0.0)

    @pl.when(k == 0)
    def _():
        ssq_ref[0] = vec

    @pl.when(k != 0)
    def _():
        ssq_ref[0] = ssq_ref[0] + vec

    @pl.when(k == kz - 1)
    def _():
        a = ssq_ref[0]
        mx = jnp.max(a)
        idx = jnp.min(jnp.where(a >= mx, lane, 8))
        win_ref[0] = jnp.zeros((1, 8), jnp.int32) + idx


def _p2_body(w_ref, x_ref, o_ref):
    b = pl.program_id(0)
    p = w_ref[b]
    pz = p // 4
    px = (p // 2) % 2
    py = p % 2
    t = x_ref[0, :, :, pz]
    c_, zc2, yy, xx = t.shape
    ixs = 2 * lax.broadcasted_iota(jnp.int32, (c_, zc2, yy, xx // 2), 3) + px
    a = jnp.take_along_axis(t, ixs, axis=3)
    a = jnp.swapaxes(a, -1, -2)
    iy = 2 * lax.broadcasted_iota(jnp.int32, (c_, zc2, xx // 2, yy // 2), 3) + py
    a = jnp.take_along_axis(a, iy, axis=3)
    o_ref[0] = jnp.swapaxes(a, -1, -2)


def kernel(input_to_pool):
    xin = input_to_pool
    B, C, Z, Y, X = xin.shape
    Zh, Yh = Z // 2, Y // 2
    x6 = xin.reshape(B, C, Zh, 2, Y, X)

    Zc = 4 if Zh % 4 == 0 else 1
    KZ = Zh // Zc

    ssq, win = pl.pallas_call(
        lambda xr, sr, wr: _p1_body(xr, sr, wr, kz=KZ),
        grid=(B, KZ),
        in_specs=[pl.BlockSpec((1, C, Zc, 2, Y, X),
                               lambda b, k: (b, 0, k, 0, 0, 0))],
        out_specs=[
            pl.BlockSpec((1, 1, 8), lambda b, k: (b, 0, 0)),
            pl.BlockSpec((1, 1, 8), lambda b, k: (b, 0, 0)),
        ],
        out_shape=[
            jax.ShapeDtypeStruct((B, 1, 8), jnp.float32),
            jax.ShapeDtypeStruct((B, 1, 8), jnp.int32),
        ],
        compiler_params=pltpu.CompilerParams(
            dimension_semantics=("parallel", "arbitrary"),
        ),
    )(x6)

    w = win[:, 0, 0]

    Zc2 = 4 if Zh % 4 == 0 else 1
    KZ2 = Zh // Zc2

    grid_spec = pltpu.PrefetchScalarGridSpec(
        num_scalar_prefetch=1,
        grid=(B, KZ2),
        in_specs=[
            pl.BlockSpec((1, C, Zc2, 2, Y, X),
                         lambda b, k, wr: (b, 0, k, 0, 0, 0)),
        ],
        out_specs=pl.BlockSpec((1, C, Zc2, Yh, X // 2),
                               lambda b, k, wr: (b, 0, k, 0, 0)),
    )
    out = pl.pallas_call(
        _p2_body,
        grid_spec=grid_spec,
        out_shape=jax.ShapeDtypeStruct((B, C, Zh, Yh, X // 2), jnp.float32),
        compiler_params=pltpu.CompilerParams(
            dimension_semantics=("parallel", "parallel"),
        ),
    )(w, x6)
    return out
